# baseline (device time: 136091 ns/iter reference)
import jax
import jax.numpy as jnp
from jax import lax
from jax.experimental import pallas as pl
from jax.experimental.pallas import tpu as pltpu

N_DEV = 16


def kernel(A, B):
    m_per, k = A.shape
    _, n = B.shape
    A = A.astype(jnp.bfloat16)
    B = B.astype(jnp.bfloat16)

    CW = 8
    CCW = N_DEV - 1 - CW

    RING = (0, 4, 8, 12, 15, 11, 7, 3, 2, 6, 10, 14, 13, 9, 5, 1)
    POS = tuple(RING.index(i) for i in range(N_DEV))

    def body(a_ref, b_ref, out_ref, ag_ref, stage_ref,
             cw_send_sems, cw_recv_sems, ccw_send_sems, ccw_recv_sems,
             copy_sems):
        my = lax.axis_index("i")

        def lut(table, idx):
            r = jnp.int32(table[0])
            for i in range(1, N_DEV):
                r = jnp.where(idx == i, jnp.int32(table[i]), r)
            return r

        p = lut(POS, my)
        left = lut(RING, lax.rem(p + N_DEV - 1, N_DEV))
        right = lut(RING, lax.rem(p + 1, N_DEV))

        barrier_sem = pltpu.get_barrier_semaphore()
        for nbr in (left, right):
            pl.semaphore_signal(
                barrier_sem, inc=1,
                device_id=(nbr,), device_id_type=pl.DeviceIdType.MESH,
            )
        pl.semaphore_wait(barrier_sem, 2)

        ag_ref[pl.ds(my * m_per, m_per), :] = a_ref[...]

        copy_state = {"count": 0, "pending": [None, None]}

        def compute_chunk(idx):
            slot = copy_state["count"] % 2
            if copy_state["count"] >= 2:
                copy_state["pending"][slot].wait()
            stage_ref[slot] = jnp.dot(
                ag_ref[pl.ds(idx * m_per, m_per), :], b_ref[...],
                preferred_element_type=jnp.float32,
            ).astype(jnp.bfloat16)
            copy = pltpu.make_async_copy(
                stage_ref.at[slot],
                out_ref.at[pl.ds(idx * m_per, m_per), :],
                copy_sems.at[slot],
            )
            copy.start()
            copy_state["pending"][slot] = copy
            copy_state["count"] += 1

        def chunk_rdma(idx, send_sem, recv_sem, dev):
            return pltpu.make_async_remote_copy(
                src_ref=ag_ref.at[pl.ds(idx * m_per, m_per), :],
                dst_ref=ag_ref.at[pl.ds(idx * m_per, m_per), :],
                send_sem=send_sem,
                recv_sem=recv_sem,
                device_id=(dev,),
                device_id_type=pl.DeviceIdType.MESH,
            )

        pending_sends = []
        for h in range(CW):
            cw_origin = lut(RING, lax.rem(p + N_DEV - h, N_DEV))
            cw_in = lut(RING, lax.rem(p + 2 * N_DEV - h - 1, N_DEV))
            send_cw = chunk_rdma(
                cw_origin, cw_send_sems.at[h], cw_recv_sems.at[h], right
            )
            send_cw.start()
            if h < CCW:
                ccw_origin = lut(RING, lax.rem(p + h, N_DEV))
                ccw_in = lut(RING, lax.rem(p + h + 1, N_DEV))
                send_ccw = chunk_rdma(
                    ccw_origin, ccw_send_sems.at[h], ccw_recv_sems.at[h], left
                )
                send_ccw.start()
            if h == 0:
                compute_chunk(my)
            else:
                compute_chunk(cw_origin)
                compute_chunk(lut(RING, lax.rem(p + h, N_DEV)))
            pending_sends.append(send_cw)
            chunk_rdma(
                cw_in, cw_send_sems.at[h], cw_recv_sems.at[h], left
            ).wait_recv()
            if h < CCW:
                pending_sends.append(send_ccw)
                chunk_rdma(
                    ccw_in, ccw_send_sems.at[h], ccw_recv_sems.at[h], right
                ).wait_recv()

        compute_chunk(lut(RING, lax.rem(p + N_DEV - CW, N_DEV)))

        for s in pending_sends:
            s.wait_send()
        copy_state["pending"][0].wait()
        copy_state["pending"][1].wait()

    return pl.pallas_call(
        body,
        out_shape=jax.ShapeDtypeStruct((N_DEV * m_per, n), jnp.bfloat16),
        in_specs=[
            pl.BlockSpec(memory_space=pltpu.VMEM),
            pl.BlockSpec(memory_space=pltpu.VMEM),
        ],
        out_specs=pl.BlockSpec(memory_space=pl.ANY),
        scratch_shapes=[
            pltpu.VMEM((N_DEV * m_per, k), jnp.bfloat16),
            pltpu.VMEM((2, m_per, n), jnp.bfloat16),
            pltpu.SemaphoreType.DMA((CW,)),
            pltpu.SemaphoreType.DMA((CW,)),
            pltpu.SemaphoreType.DMA((CCW,)),
            pltpu.SemaphoreType.DMA((CCW,)),
            pltpu.SemaphoreType.DMA((2,)),
        ],
        compiler_params=pltpu.CompilerParams(collective_id=0),
    )(A, B)


# device time: 122952 ns/iter; 1.1069x vs baseline; 1.1069x over previous
import jax
import jax.numpy as jnp
from jax import lax
from jax.experimental import pallas as pl
from jax.experimental.pallas import tpu as pltpu

N_DEV = 16


def kernel(A, B):
    m_per, k = A.shape
    _, n = B.shape
    A = A.astype(jnp.bfloat16)
    B = B.astype(jnp.bfloat16)

    CW = 8
    CCW = N_DEV - 1 - CW

    RING = (0, 4, 8, 12, 15, 11, 7, 3, 2, 6, 10, 14, 13, 9, 5, 1)
    POS = tuple(RING.index(i) for i in range(N_DEV))

    def body(a_ref, b_ref, out_ref, ag_ref, stage_ref,
             cw_send_sems, cw_recv_sems, ccw_send_sems, ccw_recv_sems,
             copy_sems):
        my = lax.axis_index("i")

        def lut(table, idx):
            r = jnp.int32(table[0])
            for i in range(1, N_DEV):
                r = jnp.where(idx == i, jnp.int32(table[i]), r)
            return r

        p = lut(POS, my)
        left = lut(RING, lax.rem(p + N_DEV - 1, N_DEV))
        right = lut(RING, lax.rem(p + 1, N_DEV))

        barrier_sem = pltpu.get_barrier_semaphore()
        for nbr in (left, right):
            pl.semaphore_signal(
                barrier_sem, inc=1,
                device_id=(nbr,), device_id_type=pl.DeviceIdType.MESH,
            )
        pl.semaphore_wait(barrier_sem, 2)

        ag_ref[pl.ds(my * m_per, m_per), :] = a_ref[...]

        copy_state = {"count": 0, "pending": [None, None]}

        def compute_chunk(idx):
            slot = copy_state["count"] % 2
            if copy_state["count"] >= 2:
                copy_state["pending"][slot].wait()
            stage_ref[slot] = jnp.dot(
                ag_ref[pl.ds(idx * m_per, m_per), :], b_ref[...],
                preferred_element_type=jnp.float32,
            ).astype(jnp.bfloat16)
            copy = pltpu.make_async_copy(
                stage_ref.at[slot],
                out_ref.at[pl.ds(idx * m_per, m_per), :],
                copy_sems.at[slot],
            )
            copy.start()
            copy_state["pending"][slot] = copy
            copy_state["count"] += 1

        half_m = m_per // 2

        def half_rdma(idx, q, sems_pair, h, dev):
            sl = pl.ds(idx * m_per + q * half_m, half_m)
            send_sem, recv_sem = sems_pair
            return pltpu.make_async_remote_copy(
                src_ref=ag_ref.at[sl, :],
                dst_ref=ag_ref.at[sl, :],
                send_sem=send_sem.at[2 * h + q],
                recv_sem=recv_sem.at[2 * h + q],
                device_id=(dev,),
                device_id_type=pl.DeviceIdType.MESH,
            )

        cw_sems = (cw_send_sems, cw_recv_sems)
        ccw_sems = (ccw_send_sems, ccw_recv_sems)
        pending_sends = []

        def start_send(idx, q, sems_pair, h, dev):
            s = half_rdma(idx, q, sems_pair, h, dev)
            s.start()
            pending_sends.append(s)

        for q in (0, 1):
            start_send(my, q, cw_sems, 0, right)
            start_send(my, q, ccw_sems, 0, left)
        compute_chunk(my)

        for h in range(1, CW):
            cw_fwd = lut(RING, lax.rem(p + N_DEV - h, N_DEV))
            ccw_fwd = lut(RING, lax.rem(p + h, N_DEV))
            for q in (0, 1):
                half_rdma(cw_fwd, q, cw_sems, h - 1, left).wait_recv()
                start_send(cw_fwd, q, cw_sems, h, right)
                half_rdma(ccw_fwd, q, ccw_sems, h - 1, right).wait_recv()
                if h < CCW:
                    start_send(ccw_fwd, q, ccw_sems, h, left)
            compute_chunk(cw_fwd)
            compute_chunk(ccw_fwd)

        last_cw = lut(RING, lax.rem(p + N_DEV - CW, N_DEV))
        for q in (0, 1):
            half_rdma(last_cw, q, cw_sems, CW - 1, left).wait_recv()
        compute_chunk(last_cw)

        for s in pending_sends:
            s.wait_send()
        for c in copy_state["pending"]:
            if c is not None:
                c.wait()

    return pl.pallas_call(
        body,
        out_shape=jax.ShapeDtypeStruct((N_DEV * m_per, n), jnp.bfloat16),
        in_specs=[
            pl.BlockSpec(memory_space=pltpu.VMEM),
            pl.BlockSpec(memory_space=pltpu.VMEM),
        ],
        out_specs=pl.BlockSpec(memory_space=pl.ANY),
        scratch_shapes=[
            pltpu.VMEM((N_DEV * m_per, k), jnp.bfloat16),
            pltpu.VMEM((2, m_per, n), jnp.bfloat16),
            pltpu.SemaphoreType.DMA((2 * CW,)),
            pltpu.SemaphoreType.DMA((2 * CW,)),
            pltpu.SemaphoreType.DMA((2 * CCW,)),
            pltpu.SemaphoreType.DMA((2 * CCW,)),
            pltpu.SemaphoreType.DMA((2,)),
        ],
        compiler_params=pltpu.CompilerParams(collective_id=0),
    )(A, B)
